# Initial kernel scaffold; baseline (speedup 1.0000x reference)
import jax
import jax.numpy as jnp
from jax.experimental import pallas as pl


def kernel(q, v, embeddings):
    raise NotImplementedError("write your pallas kernel here")


# speedup vs baseline: 8.2908x; 8.2908x over previous
import jax
import jax.numpy as jnp
from jax.experimental import pallas as pl
from jax.experimental.pallas import tpu as pltpu

_K = 8


def _body(emb_hbm, out_hbm, emb_v, b_v, sems, load_sem):
    Sq = out_hbm.shape[0]
    Sv = out_hbm.shape[1]
    n_emb = emb_hbm.shape[0]
    max_pos = (n_emb - 1) // 2

    cp = pltpu.make_async_copy(emb_hbm, emb_v, load_sem)
    cp.start()
    cp.wait()
    e = emb_v[...]

    lo = Sq - max_pos
    hi = Sq + max_pos
    b_v[0:lo, :] = jnp.broadcast_to(e[0:1, :], (lo, b_v.shape[1]))
    b_v[lo:hi, :] = e[1:n_emb, :]
    b_v[hi:, :] = jnp.broadcast_to(
        e[n_emb - 1 : n_emb, :], (b_v.shape[0] - hi, b_v.shape[1])
    )

    def row_copy(i):
        return pltpu.make_async_copy(
            b_v.at[pl.ds(Sq - 1 - i, Sv), :],
            out_hbm.at[i],
            sems.at[jax.lax.rem(i, _K)],
        )

    def body(i, carry):
        @pl.when(i >= _K)
        def _():
            row_copy(i - _K).wait()

        row_copy(i).start()
        return carry

    jax.lax.fori_loop(0, Sq, body, 0)
    for r in range(_K):
        row_copy(Sq - _K + r).wait()


def kernel(q, v, embeddings):
    Sq = q.shape[1]
    Sv = v.shape[1]
    n_emb, d = embeddings.shape
    out_shape = jax.ShapeDtypeStruct((Sq, Sv, d), embeddings.dtype)
    return pl.pallas_call(
        _body,
        out_shape=out_shape,
        in_specs=[pl.BlockSpec(memory_space=pltpu.MemorySpace.HBM)],
        out_specs=pl.BlockSpec(memory_space=pltpu.MemorySpace.HBM),
        scratch_shapes=[
            pltpu.VMEM((n_emb, d), embeddings.dtype),
            pltpu.VMEM((Sq + Sv, d), embeddings.dtype),
            pltpu.SemaphoreType.DMA((_K,)),
            pltpu.SemaphoreType.DMA,
        ],
    )(embeddings)
